# baseline (speedup 1.0000x reference)
import functools
import jax
import jax.numpy as jnp
from jax import lax
from jax.experimental import pallas as pl
from jax.experimental.pallas import tpu as pltpu
from jax.experimental.pallas import tpu_sc as plsc

N = 10000
E = 320000
IN_DIM = 128
HID = 16
HEADS = 8
F1 = HEADS * HID
OUT_DIM = 40
F2PAD = 48
ROW1 = F1 + 16
ROW2 = F2PAD

NC = 2
NS = 16
NW = NC * NS
EPW = E // NW
CH = 80
NCHUNK = EPW // CH
RPT = N // NS

_TCB = 400
_TCG = N // _TCB


def _tca_body(x_ref, w1_ref, as_ref, ad_ref, r_ref, h_ref, p_ref, q_ref,
              self_ref):
    h = jnp.dot(x_ref[...], w1_ref[...], preferred_element_type=jnp.float32)
    a_s = jnp.dot(h, as_ref[...], preferred_element_type=jnp.float32)
    a_d = jnp.dot(h, ad_ref[...], preferred_element_type=jnp.float32)
    z = a_s + a_d
    ws = jnp.exp(jnp.where(z >= 0, z, 0.2 * z))
    numself = h * jnp.dot(ws, r_ref[...], preferred_element_type=jnp.float32)
    h_ref[...] = h
    p_ref[...] = jnp.concatenate([a_s, a_s], axis=1)
    q_ref[...] = jnp.concatenate([a_d, a_d], axis=1)
    self_ref[...] = jnp.concatenate([numself, ws, ws], axis=1)


def _tca(x, w1, as_mat, ad_mat, rep):
    return pl.pallas_call(
        _tca_body,
        grid=(_TCG,),
        in_specs=[
            pl.BlockSpec((_TCB, IN_DIM), lambda i: (i, 0)),
            pl.BlockSpec((IN_DIM, F1), lambda i: (0, 0)),
            pl.BlockSpec((F1, HEADS), lambda i: (0, 0)),
            pl.BlockSpec((F1, HEADS), lambda i: (0, 0)),
            pl.BlockSpec((HEADS, F1), lambda i: (0, 0)),
        ],
        out_specs=[
            pl.BlockSpec((_TCB, F1), lambda i: (i, 0)),
            pl.BlockSpec((_TCB, 16), lambda i: (i, 0)),
            pl.BlockSpec((_TCB, 16), lambda i: (i, 0)),
            pl.BlockSpec((_TCB, ROW1), lambda i: (i, 0)),
        ],
        out_shape=[
            jax.ShapeDtypeStruct((N, F1), jnp.float32),
            jax.ShapeDtypeStruct((N, 16), jnp.float32),
            jax.ShapeDtypeStruct((N, 16), jnp.float32),
            jax.ShapeDtypeStruct((N, ROW1), jnp.float32),
        ],
    )(x, w1, as_mat, ad_mat, rep)


def _sc1_body(src_hbm, dst_hbm, p_hbm, q_hbm, h_hbm, out_hbm,
              sidx0, didx0, pbuf0, qbuf0, hbuf0,
              sidx1, didx1, pbuf1, qbuf1, hbuf1,
              sbuf0, acc, sem0, sem1):
    c = lax.axis_index("c")
    s = lax.axis_index("s")
    wid = c * NS + s
    B0 = (sidx0, didx0, pbuf0, qbuf0, hbuf0, sbuf0)
    B1 = (sidx1, didx1, pbuf1, qbuf1, hbuf1, sbuf0)

    def _zrow(r, _):
        for j in range(ROW1 // 16):
            sbuf0[r, pl.ds(j * 16, 16)] = jnp.zeros((16,), jnp.float32)
        return 0
    lax.fori_loop(0, CH, _zrow, 0)
    rbase = s * RPT
    for k in range(RPT // CH):
        pltpu.sync_copy(sbuf0, acc.at[pl.ds(rbase + k * CH, CH)])
    rem = RPT - (RPT // CH) * CH
    if rem:
        pltpu.sync_copy(sbuf0.at[pl.ds(0, rem)],
                        acc.at[pl.ds(rbase + (RPT // CH) * CH, rem)])

    def _issue(bufs, sem, b):
        si, di, pb, qb, hb, _ = bufs
        pltpu.sync_copy(src_hbm.at[pl.ds(b, CH)], si)
        pltpu.sync_copy(dst_hbm.at[pl.ds(b, CH)], di)
        pltpu.async_copy(p_hbm.at[si], pb, sem)
        pltpu.async_copy(q_hbm.at[di], qb, sem)
        pltpu.async_copy(h_hbm.at[si], hb, sem)

    def _wait(bufs, sem):
        si, di, pb, qb, hb, _ = bufs
        pltpu.make_async_copy(p_hbm.at[si], pb, sem).wait()
        pltpu.make_async_copy(q_hbm.at[di], qb, sem).wait()
        pltpu.make_async_copy(h_hbm.at[si], hb, sem).wait()

    def _compute(bufs):
        _, _, pb, qb, hb, sb = bufs

        def _edge(e, _):
            z = pb[e] + qb[e]
            w = jnp.exp(jnp.where(z >= 0, z, 0.2 * z))
            sb[e, pl.ds(F1, 16)] = w
            for hh in range(HEADS):
                wb = w.at[jnp.full((16,), hh, jnp.int32)].get(mode="promise_in_bounds")
                sb[e, pl.ds(hh * HID, HID)] = (
                    wb * hb[e, pl.ds(hh * HID, HID)])
            return 0
        lax.fori_loop(0, CH, _edge, 0, unroll=2)

    def _scatter(bufs):
        _, di, _, _, _, sb = bufs
        pltpu.sync_copy(sb, acc.at[di], add=True)

    ebase = wid * EPW
    _issue(B0, sem0, ebase)
    plsc.subcore_barrier()

    def _pair(k, _):
        b = ebase + (2 * k) * CH
        _wait(B0, sem0)
        _issue(B1, sem1, b + CH)
        _compute(B0)
        _scatter(B0)
        _wait(B1, sem1)
        _issue(B0, sem0, b + 2 * CH)
        _compute(B1)
        _scatter(B1)
        return 0
    lax.fori_loop(0, (NCHUNK - 1) // 2, _pair, 0)
    _wait(B0, sem0)
    _compute(B0)
    _scatter(B0)

    plsc.subcore_barrier()
    for k in range(RPT // 125):
        r0 = rbase + k * 125
        pltpu.sync_copy(acc.at[pl.ds(r0, 125)], out_hbm.at[c, pl.ds(r0, 125)])


def _sc1(src, dst, p, q, h):
    mesh = plsc.VectorSubcoreMesh(core_axis_name="c", subcore_axis_name="s")
    f = functools.partial(
        pl.kernel,
        out_type=jax.ShapeDtypeStruct((NC, N, ROW1), jnp.float32),
        mesh=mesh,
        scratch_types=(
            2 * [
                pltpu.VMEM((CH,), jnp.int32),
                pltpu.VMEM((CH,), jnp.int32),
                pltpu.VMEM((CH, 16), jnp.float32),
                pltpu.VMEM((CH, 16), jnp.float32),
                pltpu.VMEM((CH, F1), jnp.float32),
            ] + [
                pltpu.VMEM((CH, ROW1), jnp.float32),
                pltpu.VMEM_SHARED((N, ROW1), jnp.float32),
                pltpu.SemaphoreType.DMA,
                pltpu.SemaphoreType.DMA,
            ]
        ),
        compiler_params=pltpu.CompilerParams(use_tc_tiling_on_sc=False, needs_layout_passes=False),
    )(_sc1_body)
    return f(src, dst, p, q, h)


def _tcb_body(acc_ref, self_ref, b1_ref, w2_ref, as2_ref, ad2_ref, r_ref,
              h2_ref, asv_ref, adv_ref, self2_ref):
    tot = acc_ref[0] + acc_ref[1] + self_ref[...]
    num = tot[:, :F1]
    den = tot[:, F1:F1 + HEADS]
    den128 = jnp.dot(den, r_ref[...], preferred_element_type=jnp.float32)
    h1 = num / (den128 + 1e-16) + b1_ref[...]
    h1 = jnp.where(h1 > 0, h1, jnp.exp(h1) - 1.0)
    h2 = jnp.dot(h1, w2_ref[...], preferred_element_type=jnp.float32)
    a_s = jnp.dot(h2, as2_ref[...], preferred_element_type=jnp.float32)
    a_d = jnp.dot(h2, ad2_ref[...], preferred_element_type=jnp.float32)
    z = a_s + a_d
    ws = jnp.exp(jnp.where(z >= 0, z, 0.2 * z))
    opad = jnp.ones((_TCB, F2PAD - OUT_DIM), jnp.float32)
    hp = jnp.concatenate([h2, opad], axis=1)
    h2_ref[...] = hp
    asv_ref[...] = a_s
    adv_ref[...] = a_d
    self2_ref[...] = hp * ws


def _tcb(acc, selfacc, b1, w2, as2, ad2, rep):
    return pl.pallas_call(
        _tcb_body,
        grid=(_TCG,),
        in_specs=[
            pl.BlockSpec((NC, _TCB, ROW1), lambda i: (0, i, 0)),
            pl.BlockSpec((_TCB, ROW1), lambda i: (i, 0)),
            pl.BlockSpec((1, F1), lambda i: (0, 0)),
            pl.BlockSpec((F1, OUT_DIM), lambda i: (0, 0)),
            pl.BlockSpec((OUT_DIM, 1), lambda i: (0, 0)),
            pl.BlockSpec((OUT_DIM, 1), lambda i: (0, 0)),
            pl.BlockSpec((HEADS, F1), lambda i: (0, 0)),
        ],
        out_specs=[
            pl.BlockSpec((_TCB, F2PAD), lambda i: (i, 0)),
            pl.BlockSpec((_TCB, 1), lambda i: (i, 0)),
            pl.BlockSpec((_TCB, 1), lambda i: (i, 0)),
            pl.BlockSpec((_TCB, ROW2), lambda i: (i, 0)),
        ],
        out_shape=[
            jax.ShapeDtypeStruct((N, F2PAD), jnp.float32),
            jax.ShapeDtypeStruct((N, 1), jnp.float32),
            jax.ShapeDtypeStruct((N, 1), jnp.float32),
            jax.ShapeDtypeStruct((N, ROW2), jnp.float32),
        ],
    )(acc, selfacc, b1, w2, as2, ad2, rep)


def _sc2_body(src_hbm, dst_hbm, h2_hbm, as_hbm, ad_hbm, out_hbm,
              sidx0, didx0, hbuf0, sbuf0,
              sidx1, didx1, hbuf1, sbuf1,
              wbuf, asb, adb, acc, sem0, sem1):
    c = lax.axis_index("c")
    s = lax.axis_index("s")
    wid = c * NS + s
    B0 = (sidx0, didx0, hbuf0, sbuf0)
    B1 = (sidx1, didx1, hbuf1, sbuf1)

    def _zrow(r, _):
        for j in range(ROW2 // 16):
            sbuf0[r, pl.ds(j * 16, 16)] = jnp.zeros((16,), jnp.float32)
        return 0
    lax.fori_loop(0, CH, _zrow, 0)
    rbase = s * RPT
    for k in range(RPT // CH):
        pltpu.sync_copy(sbuf0, acc.at[pl.ds(rbase + k * CH, CH)])
    rem = RPT - (RPT // CH) * CH
    if rem:
        pltpu.sync_copy(sbuf0.at[pl.ds(0, rem)],
                        acc.at[pl.ds(rbase + (RPT // CH) * CH, rem)])

    pltpu.sync_copy(as_hbm, asb)
    pltpu.sync_copy(ad_hbm, adb)

    def _issue(bufs, sem, b):
        si, di, hb, _ = bufs
        pltpu.sync_copy(src_hbm.at[pl.ds(b, CH)], si)
        pltpu.sync_copy(dst_hbm.at[pl.ds(b, CH)], di)
        pltpu.async_copy(h2_hbm.at[si], hb, sem)

    def _wait(bufs, sem):
        si, _, hb, _ = bufs
        pltpu.make_async_copy(h2_hbm.at[si], hb, sem).wait()

    def _compute(bufs):
        si, di, hb, sb = bufs

        def _att(k, _):
            sv = si[pl.ds(k * 16, 16)]
            dv = di[pl.ds(k * 16, 16)]
            z = plsc.load_gather(asb, [sv]) + plsc.load_gather(adb, [dv])
            wbuf[pl.ds(k * 16, 16)] = jnp.exp(jnp.where(z >= 0, z, 0.2 * z))
            return 0
        lax.fori_loop(0, CH // 16, _att, 0)

        def _grp(k, _):
            w16 = wbuf[pl.ds(k * 16, 16)]
            for j in range(16):
                e = k * 16 + j
                wb = w16.at[jnp.full((16,), j, jnp.int32)].get(mode="promise_in_bounds")
                for t in range(F2PAD // 16):
                    sb[e, pl.ds(t * 16, 16)] = (
                        wb * hb[e, pl.ds(t * 16, 16)])
            return 0
        lax.fori_loop(0, CH // 16, _grp, 0)

    def _scatter(bufs):
        _, di, _, sb = bufs
        pltpu.sync_copy(sb, acc.at[di], add=True)

    ebase = wid * EPW
    _issue(B0, sem0, ebase)
    plsc.subcore_barrier()

    def _pair(k, _):
        b = ebase + (2 * k) * CH
        _wait(B0, sem0)
        _issue(B1, sem1, b + CH)
        _compute(B0)
        _scatter(B0)
        _wait(B1, sem1)
        _issue(B0, sem0, b + 2 * CH)
        _compute(B1)
        _scatter(B1)
        return 0
    lax.fori_loop(0, (NCHUNK - 1) // 2, _pair, 0)
    _wait(B0, sem0)
    _compute(B0)
    _scatter(B0)

    plsc.subcore_barrier()
    for k in range(RPT // 125):
        r0 = rbase + k * 125
        pltpu.sync_copy(acc.at[pl.ds(r0, 125)], out_hbm.at[c, pl.ds(r0, 125)])


def _sc2(src, dst, h2, as2, ad2):
    mesh = plsc.VectorSubcoreMesh(core_axis_name="c", subcore_axis_name="s")
    f = functools.partial(
        pl.kernel,
        out_type=jax.ShapeDtypeStruct((NC, N, ROW2), jnp.float32),
        mesh=mesh,
        scratch_types=(
            2 * [
                pltpu.VMEM((CH,), jnp.int32),
                pltpu.VMEM((CH,), jnp.int32),
                pltpu.VMEM((CH, F2PAD), jnp.float32),
                pltpu.VMEM((CH, ROW2), jnp.float32),
            ] + [
                pltpu.VMEM((CH,), jnp.float32),
                pltpu.VMEM((N,), jnp.float32),
                pltpu.VMEM((N,), jnp.float32),
                pltpu.VMEM_SHARED((N, ROW2), jnp.float32),
                pltpu.SemaphoreType.DMA,
                pltpu.SemaphoreType.DMA,
            ]
        ),
        compiler_params=pltpu.CompilerParams(use_tc_tiling_on_sc=False, needs_layout_passes=False),
    )(_sc2_body)
    return f(src, dst, h2, as2, ad2)


def _tcc_body(acc_ref, self_ref, b2_ref, out_ref):
    tot = acc_ref[0] + acc_ref[1] + self_ref[...]
    num = tot[:, :OUT_DIM]
    den = tot[:, OUT_DIM:OUT_DIM + 1]
    out_ref[...] = num / (den + 1e-16) + b2_ref[...]


def _tcc(acc, selfacc, b2):
    return pl.pallas_call(
        _tcc_body,
        grid=(_TCG,),
        in_specs=[
            pl.BlockSpec((NC, _TCB, ROW2), lambda i: (0, i, 0)),
            pl.BlockSpec((_TCB, ROW2), lambda i: (i, 0)),
            pl.BlockSpec((1, OUT_DIM), lambda i: (0, 0)),
        ],
        out_specs=pl.BlockSpec((_TCB, OUT_DIM), lambda i: (i, 0)),
        out_shape=jax.ShapeDtypeStruct((N, OUT_DIM), jnp.float32),
    )(acc, selfacc, b2)


@jax.jit
def kernel(x, edge_index, W1, a_src1, a_dst1, b1, W2, a_src2, a_dst2, b2):
    src = edge_index[0]
    dst = edge_index[1]

    eye = jnp.eye(HEADS, dtype=jnp.float32)
    as_mat = (eye[:, None, :] * a_src1[:, :, None]).reshape(F1, HEADS)
    ad_mat = (eye[:, None, :] * a_dst1[:, :, None]).reshape(F1, HEADS)
    rep = jnp.repeat(eye, HID, axis=1)

    h, p, q, selfacc = _tca(x, W1, as_mat, ad_mat, rep)
    acc1 = _sc1(src, dst, p, q, h)
    h2, asv, adv, selfacc2 = _tcb(acc1, selfacc, b1.reshape(1, F1), W2,
                                  a_src2.reshape(OUT_DIM, 1),
                                  a_dst2.reshape(OUT_DIM, 1), rep)
    acc2 = _sc2(src, dst, h2, asv.reshape(N), adv.reshape(N))
    return _tcc(acc2, selfacc2, b2.reshape(1, OUT_DIM))


# speedup vs baseline: 1.6118x; 1.6118x over previous
import functools
import jax
import jax.numpy as jnp
from jax import lax
from jax.experimental import pallas as pl
from jax.experimental.pallas import tpu as pltpu
from jax.experimental.pallas import tpu_sc as plsc

N = 10000
E = 320000
IN_DIM = 128
HID = 16
HEADS = 8
F1 = HEADS * HID
OUT_DIM = 40
F2PAD = 48
ROW1 = F1 + 16
ROW2 = F2PAD

NC = 2
NS = 16
NW = NC * NS
EPW = E // NW
CH = 80
NCHUNK = EPW // CH
RPT = N // NS

_TCB = 400
_TCG = N // _TCB


def _tca_body(x_ref, w1_ref, as_ref, ad_ref, r_ref, h_ref, p_ref, q_ref,
              self_ref):
    h = jnp.dot(x_ref[...], w1_ref[...], preferred_element_type=jnp.float32)
    a_s = jnp.dot(h, as_ref[...], preferred_element_type=jnp.float32)
    a_d = jnp.dot(h, ad_ref[...], preferred_element_type=jnp.float32)
    z = a_s + a_d
    ws = jnp.exp(jnp.where(z >= 0, z, 0.2 * z))
    numself = h * jnp.dot(ws, r_ref[...], preferred_element_type=jnp.float32)
    h_ref[...] = h
    p_ref[...] = jnp.concatenate([a_s, a_s], axis=1)
    q_ref[...] = jnp.concatenate([a_d, a_d], axis=1)
    self_ref[...] = jnp.concatenate([numself, ws, ws], axis=1)


def _tca(x, w1, as_mat, ad_mat, rep):
    return pl.pallas_call(
        _tca_body,
        grid=(_TCG,),
        in_specs=[
            pl.BlockSpec((_TCB, IN_DIM), lambda i: (i, 0)),
            pl.BlockSpec((IN_DIM, F1), lambda i: (0, 0)),
            pl.BlockSpec((F1, HEADS), lambda i: (0, 0)),
            pl.BlockSpec((F1, HEADS), lambda i: (0, 0)),
            pl.BlockSpec((HEADS, F1), lambda i: (0, 0)),
        ],
        out_specs=[
            pl.BlockSpec((_TCB, F1), lambda i: (i, 0)),
            pl.BlockSpec((_TCB, 16), lambda i: (i, 0)),
            pl.BlockSpec((_TCB, 16), lambda i: (i, 0)),
            pl.BlockSpec((_TCB, ROW1), lambda i: (i, 0)),
        ],
        out_shape=[
            jax.ShapeDtypeStruct((N, F1), jnp.float32),
            jax.ShapeDtypeStruct((N, 16), jnp.float32),
            jax.ShapeDtypeStruct((N, 16), jnp.float32),
            jax.ShapeDtypeStruct((N, ROW1), jnp.float32),
        ],
    )(x, w1, as_mat, ad_mat, rep)


def _sc1_body(src_hbm, dst_hbm, p_hbm, q_hbm, h_hbm, out_hbm,
              sidx0, didx0, pbuf0, qbuf0, hbuf0,
              sidx1, didx1, pbuf1, qbuf1, hbuf1,
              sbuf0, acc, sem0, sem1):
    c = lax.axis_index("c")
    s = lax.axis_index("s")
    wid = c * NS + s
    B0 = (sidx0, didx0, pbuf0, qbuf0, hbuf0, sbuf0)
    B1 = (sidx1, didx1, pbuf1, qbuf1, hbuf1, sbuf0)

    def _zrow(r, _):
        for j in range(ROW1 // 16):
            sbuf0[r, pl.ds(j * 16, 16)] = jnp.zeros((16,), jnp.float32)
        return 0
    lax.fori_loop(0, CH, _zrow, 0)
    rbase = s * RPT
    for k in range(RPT // CH):
        pltpu.sync_copy(sbuf0, acc.at[pl.ds(rbase + k * CH, CH)])
    rem = RPT - (RPT // CH) * CH
    if rem:
        pltpu.sync_copy(sbuf0.at[pl.ds(0, rem)],
                        acc.at[pl.ds(rbase + (RPT // CH) * CH, rem)])

    def _issue(bufs, sem, b):
        si, di, pb, qb, hb, _ = bufs
        pltpu.sync_copy(src_hbm.at[pl.ds(b, CH)], si)
        pltpu.sync_copy(dst_hbm.at[pl.ds(b, CH)], di)
        pltpu.async_copy(p_hbm.at[si], pb, sem)
        pltpu.async_copy(q_hbm.at[di], qb, sem)
        pltpu.async_copy(h_hbm.at[si], hb, sem)

    def _wait(bufs, sem):
        si, di, pb, qb, hb, _ = bufs
        pltpu.make_async_copy(p_hbm.at[si], pb, sem).wait()
        pltpu.make_async_copy(q_hbm.at[di], qb, sem).wait()
        pltpu.make_async_copy(h_hbm.at[si], hb, sem).wait()

    def _compute(bufs):
        _, _, pb, qb, hb, sb = bufs

        @plsc.parallel_loop(0, CH, unroll=2)
        def _edge(e):
            z = pb[e] + qb[e]
            w = jnp.exp(jnp.where(z >= 0, z, 0.2 * z))
            sb[e, pl.ds(F1, 16)] = w
            for hh in range(HEADS):
                wb = w.at[jnp.full((16,), hh, jnp.int32)].get(
                    mode="promise_in_bounds")
                sb[e, pl.ds(hh * HID, HID)] = (
                    wb * hb[e, pl.ds(hh * HID, HID)])

    def _scatter(bufs):
        _, di, _, _, _, sb = bufs
        pltpu.sync_copy(sb, acc.at[di], add=True)

    ebase = wid * EPW
    _issue(B0, sem0, ebase)
    plsc.subcore_barrier()

    def _pair(k, _):
        b = ebase + (2 * k) * CH
        _wait(B0, sem0)
        _issue(B1, sem1, b + CH)
        _compute(B0)
        _scatter(B0)
        _wait(B1, sem1)
        _issue(B0, sem0, b + 2 * CH)
        _compute(B1)
        _scatter(B1)
        return 0
    lax.fori_loop(0, (NCHUNK - 1) // 2, _pair, 0)
    _wait(B0, sem0)
    _compute(B0)
    _scatter(B0)

    plsc.subcore_barrier()
    for k in range(RPT // 125):
        r0 = rbase + k * 125
        pltpu.sync_copy(acc.at[pl.ds(r0, 125)], out_hbm.at[c, pl.ds(r0, 125)])


def _sc1(src, dst, p, q, h):
    mesh = plsc.VectorSubcoreMesh(core_axis_name="c", subcore_axis_name="s")
    f = functools.partial(
        pl.kernel,
        out_type=jax.ShapeDtypeStruct((NC, N, ROW1), jnp.float32),
        mesh=mesh,
        scratch_types=(
            2 * [
                pltpu.VMEM((CH,), jnp.int32),
                pltpu.VMEM((CH,), jnp.int32),
                pltpu.VMEM((CH, 16), jnp.float32),
                pltpu.VMEM((CH, 16), jnp.float32),
                pltpu.VMEM((CH, F1), jnp.float32),
            ] + [
                pltpu.VMEM((CH, ROW1), jnp.float32),
                pltpu.VMEM_SHARED((N, ROW1), jnp.float32),
                pltpu.SemaphoreType.DMA,
                pltpu.SemaphoreType.DMA,
            ]
        ),
        compiler_params=pltpu.CompilerParams(use_tc_tiling_on_sc=False, needs_layout_passes=False),
    )(_sc1_body)
    return f(src, dst, p, q, h)


def _tcb_body(acc_ref, self_ref, b1_ref, w2_ref, as2_ref, ad2_ref, r_ref,
              h2_ref, asv_ref, adv_ref, self2_ref):
    tot = acc_ref[0] + acc_ref[1] + self_ref[...]
    num = tot[:, :F1]
    den = tot[:, F1:F1 + HEADS]
    den128 = jnp.dot(den, r_ref[...], preferred_element_type=jnp.float32)
    h1 = num / (den128 + 1e-16) + b1_ref[...]
    h1 = jnp.where(h1 > 0, h1, jnp.exp(h1) - 1.0)
    h2 = jnp.dot(h1, w2_ref[...], preferred_element_type=jnp.float32)
    a_s = jnp.dot(h2, as2_ref[...], preferred_element_type=jnp.float32)
    a_d = jnp.dot(h2, ad2_ref[...], preferred_element_type=jnp.float32)
    z = a_s + a_d
    ws = jnp.exp(jnp.where(z >= 0, z, 0.2 * z))
    opad = jnp.ones((_TCB, F2PAD - OUT_DIM), jnp.float32)
    hp = jnp.concatenate([h2, opad], axis=1)
    h2_ref[...] = hp
    asv_ref[...] = a_s
    adv_ref[...] = a_d
    self2_ref[...] = hp * ws


def _tcb(acc, selfacc, b1, w2, as2, ad2, rep):
    return pl.pallas_call(
        _tcb_body,
        grid=(_TCG,),
        in_specs=[
            pl.BlockSpec((NC, _TCB, ROW1), lambda i: (0, i, 0)),
            pl.BlockSpec((_TCB, ROW1), lambda i: (i, 0)),
            pl.BlockSpec((1, F1), lambda i: (0, 0)),
            pl.BlockSpec((F1, OUT_DIM), lambda i: (0, 0)),
            pl.BlockSpec((OUT_DIM, 1), lambda i: (0, 0)),
            pl.BlockSpec((OUT_DIM, 1), lambda i: (0, 0)),
            pl.BlockSpec((HEADS, F1), lambda i: (0, 0)),
        ],
        out_specs=[
            pl.BlockSpec((_TCB, F2PAD), lambda i: (i, 0)),
            pl.BlockSpec((_TCB, 1), lambda i: (i, 0)),
            pl.BlockSpec((_TCB, 1), lambda i: (i, 0)),
            pl.BlockSpec((_TCB, ROW2), lambda i: (i, 0)),
        ],
        out_shape=[
            jax.ShapeDtypeStruct((N, F2PAD), jnp.float32),
            jax.ShapeDtypeStruct((N, 1), jnp.float32),
            jax.ShapeDtypeStruct((N, 1), jnp.float32),
            jax.ShapeDtypeStruct((N, ROW2), jnp.float32),
        ],
    )(acc, selfacc, b1, w2, as2, ad2, rep)


def _sc2_body(src_hbm, dst_hbm, h2_hbm, as_hbm, ad_hbm, out_hbm,
              sidx0, didx0, hbuf0, sbuf0,
              sidx1, didx1, hbuf1, sbuf1,
              wbuf, asb, adb, acc, sem0, sem1):
    c = lax.axis_index("c")
    s = lax.axis_index("s")
    wid = c * NS + s
    B0 = (sidx0, didx0, hbuf0, sbuf0)
    B1 = (sidx1, didx1, hbuf1, sbuf1)

    def _zrow(r, _):
        for j in range(ROW2 // 16):
            sbuf0[r, pl.ds(j * 16, 16)] = jnp.zeros((16,), jnp.float32)
        return 0
    lax.fori_loop(0, CH, _zrow, 0)
    rbase = s * RPT
    for k in range(RPT // CH):
        pltpu.sync_copy(sbuf0, acc.at[pl.ds(rbase + k * CH, CH)])
    rem = RPT - (RPT // CH) * CH
    if rem:
        pltpu.sync_copy(sbuf0.at[pl.ds(0, rem)],
                        acc.at[pl.ds(rbase + (RPT // CH) * CH, rem)])

    pltpu.sync_copy(as_hbm, asb)
    pltpu.sync_copy(ad_hbm, adb)

    def _issue(bufs, sem, b):
        si, di, hb, _ = bufs
        pltpu.sync_copy(src_hbm.at[pl.ds(b, CH)], si)
        pltpu.sync_copy(dst_hbm.at[pl.ds(b, CH)], di)
        pltpu.async_copy(h2_hbm.at[si], hb, sem)

    def _wait(bufs, sem):
        si, _, hb, _ = bufs
        pltpu.make_async_copy(h2_hbm.at[si], hb, sem).wait()

    def _compute(bufs):
        si, di, hb, sb = bufs

        @plsc.parallel_loop(0, CH // 16, unroll=2)
        def _att(k):
            sv = si[pl.ds(k * 16, 16)]
            dv = di[pl.ds(k * 16, 16)]
            z = plsc.load_gather(asb, [sv]) + plsc.load_gather(adb, [dv])
            wbuf[pl.ds(k * 16, 16)] = jnp.exp(jnp.where(z >= 0, z, 0.2 * z))

        @plsc.parallel_loop(0, CH // 16, unroll=2)
        def _grp(k):
            w16 = wbuf[pl.ds(k * 16, 16)]
            for j in range(16):
                e = k * 16 + j
                wb = w16.at[jnp.full((16,), j, jnp.int32)].get(
                    mode="promise_in_bounds")
                for t in range(F2PAD // 16):
                    sb[e, pl.ds(t * 16, 16)] = (
                        wb * hb[e, pl.ds(t * 16, 16)])

    def _scatter(bufs):
        _, di, _, sb = bufs
        pltpu.sync_copy(sb, acc.at[di], add=True)

    ebase = wid * EPW
    _issue(B0, sem0, ebase)
    plsc.subcore_barrier()

    def _pair(k, _):
        b = ebase + (2 * k) * CH
        _wait(B0, sem0)
        _issue(B1, sem1, b + CH)
        _compute(B0)
        _scatter(B0)
        _wait(B1, sem1)
        _issue(B0, sem0, b + 2 * CH)
        _compute(B1)
        _scatter(B1)
        return 0
    lax.fori_loop(0, (NCHUNK - 1) // 2, _pair, 0)
    _wait(B0, sem0)
    _compute(B0)
    _scatter(B0)

    plsc.subcore_barrier()
    for k in range(RPT // 125):
        r0 = rbase + k * 125
        pltpu.sync_copy(acc.at[pl.ds(r0, 125)], out_hbm.at[c, pl.ds(r0, 125)])


def _sc2(src, dst, h2, as2, ad2):
    mesh = plsc.VectorSubcoreMesh(core_axis_name="c", subcore_axis_name="s")
    f = functools.partial(
        pl.kernel,
        out_type=jax.ShapeDtypeStruct((NC, N, ROW2), jnp.float32),
        mesh=mesh,
        scratch_types=(
            2 * [
                pltpu.VMEM((CH,), jnp.int32),
                pltpu.VMEM((CH,), jnp.int32),
                pltpu.VMEM((CH, F2PAD), jnp.float32),
                pltpu.VMEM((CH, ROW2), jnp.float32),
            ] + [
                pltpu.VMEM((CH,), jnp.float32),
                pltpu.VMEM((N,), jnp.float32),
                pltpu.VMEM((N,), jnp.float32),
                pltpu.VMEM_SHARED((N, ROW2), jnp.float32),
                pltpu.SemaphoreType.DMA,
                pltpu.SemaphoreType.DMA,
            ]
        ),
        compiler_params=pltpu.CompilerParams(use_tc_tiling_on_sc=False, needs_layout_passes=False),
    )(_sc2_body)
    return f(src, dst, h2, as2, ad2)


def _tcc_body(acc_ref, self_ref, b2_ref, out_ref):
    tot = acc_ref[0] + acc_ref[1] + self_ref[...]
    num = tot[:, :OUT_DIM]
    den = tot[:, OUT_DIM:OUT_DIM + 1]
    out_ref[...] = num / (den + 1e-16) + b2_ref[...]


def _tcc(acc, selfacc, b2):
    return pl.pallas_call(
        _tcc_body,
        grid=(_TCG,),
        in_specs=[
            pl.BlockSpec((NC, _TCB, ROW2), lambda i: (0, i, 0)),
            pl.BlockSpec((_TCB, ROW2), lambda i: (i, 0)),
            pl.BlockSpec((1, OUT_DIM), lambda i: (0, 0)),
        ],
        out_specs=pl.BlockSpec((_TCB, OUT_DIM), lambda i: (i, 0)),
        out_shape=jax.ShapeDtypeStruct((N, OUT_DIM), jnp.float32),
    )(acc, selfacc, b2)


@jax.jit
def kernel(x, edge_index, W1, a_src1, a_dst1, b1, W2, a_src2, a_dst2, b2):
    src = edge_index[0]
    dst = edge_index[1]

    eye = jnp.eye(HEADS, dtype=jnp.float32)
    as_mat = (eye[:, None, :] * a_src1[:, :, None]).reshape(F1, HEADS)
    ad_mat = (eye[:, None, :] * a_dst1[:, :, None]).reshape(F1, HEADS)
    rep = jnp.repeat(eye, HID, axis=1)

    h, p, q, selfacc = _tca(x, W1, as_mat, ad_mat, rep)
    acc1 = _sc1(src, dst, p, q, h)
    h2, asv, adv, selfacc2 = _tcb(acc1, selfacc, b1.reshape(1, F1), W2,
                                  a_src2.reshape(OUT_DIM, 1),
                                  a_dst2.reshape(OUT_DIM, 1), rep)
    acc2 = _sc2(src, dst, h2, asv.reshape(N), adv.reshape(N))
    return _tcc(acc2, selfacc2, b2.reshape(1, OUT_DIM))


# speedup vs baseline: 2.2667x; 1.4064x over previous
import functools
import jax
import jax.numpy as jnp
from jax import lax
from jax.experimental import pallas as pl
from jax.experimental.pallas import tpu as pltpu
from jax.experimental.pallas import tpu_sc as plsc

N = 10000
E = 320000
IN_DIM = 128
HID = 16
HEADS = 8
F1 = HEADS * HID
OUT_DIM = 40
F2PAD = 48
ROW1 = F1 + 16
ROW2 = F2PAD

NC = 2
NS = 16
NW = NC * NS
EPW = E // NW
CH = 80
NCHUNK = EPW // CH
RPT = N // NS

_TCB = 400
_TCG = N // _TCB


def _tca_body(x_ref, w1_ref, as_ref, ad_ref, r_ref, h_ref, p_ref, q_ref,
              self_ref):
    h = jnp.dot(x_ref[...], w1_ref[...], preferred_element_type=jnp.float32)
    a_s = jnp.dot(h, as_ref[...], preferred_element_type=jnp.float32)
    a_d = jnp.dot(h, ad_ref[...], preferred_element_type=jnp.float32)
    z = a_s + a_d
    ws = jnp.exp(jnp.where(z >= 0, z, 0.2 * z))
    numself = h * jnp.dot(ws, r_ref[...], preferred_element_type=jnp.float32)
    h_ref[...] = h
    p_ref[...] = jnp.concatenate([a_s, a_s], axis=1)
    q_ref[...] = jnp.concatenate([a_d, a_d], axis=1)
    self_ref[...] = jnp.concatenate([numself, ws, ws], axis=1)


def _tca(x, w1, as_mat, ad_mat, rep):
    return pl.pallas_call(
        _tca_body,
        grid=(_TCG,),
        in_specs=[
            pl.BlockSpec((_TCB, IN_DIM), lambda i: (i, 0)),
            pl.BlockSpec((IN_DIM, F1), lambda i: (0, 0)),
            pl.BlockSpec((F1, HEADS), lambda i: (0, 0)),
            pl.BlockSpec((F1, HEADS), lambda i: (0, 0)),
            pl.BlockSpec((HEADS, F1), lambda i: (0, 0)),
        ],
        out_specs=[
            pl.BlockSpec((_TCB, F1), lambda i: (i, 0)),
            pl.BlockSpec((_TCB, 16), lambda i: (i, 0)),
            pl.BlockSpec((_TCB, 16), lambda i: (i, 0)),
            pl.BlockSpec((_TCB, ROW1), lambda i: (i, 0)),
        ],
        out_shape=[
            jax.ShapeDtypeStruct((N, F1), jnp.float32),
            jax.ShapeDtypeStruct((N, 16), jnp.float32),
            jax.ShapeDtypeStruct((N, 16), jnp.float32),
            jax.ShapeDtypeStruct((N, ROW1), jnp.float32),
        ],
    )(x, w1, as_mat, ad_mat, rep)


def _sc1_body(src_hbm, dst_hbm, p_hbm, q_hbm, h_hbm, out_hbm,
              sidx0, didx0, pbuf0, qbuf0, hbuf0,
              sidx1, didx1, pbuf1, qbuf1, hbuf1,
              sbuf0, acc, sem0, sem1, semi0, semi1):
    c = lax.axis_index("c")
    s = lax.axis_index("s")
    wid = c * NS + s
    B0 = (sidx0, didx0, pbuf0, qbuf0, hbuf0, sbuf0)
    B1 = (sidx1, didx1, pbuf1, qbuf1, hbuf1, sbuf0)

    def _zrow(r, _):
        for j in range(ROW1 // 16):
            sbuf0[r, pl.ds(j * 16, 16)] = jnp.zeros((16,), jnp.float32)
        return 0
    lax.fori_loop(0, CH, _zrow, 0)
    rbase = s * RPT
    for k in range(RPT // CH):
        pltpu.sync_copy(sbuf0, acc.at[pl.ds(rbase + k * CH, CH)])
    rem = RPT - (RPT // CH) * CH
    if rem:
        pltpu.sync_copy(sbuf0.at[pl.ds(0, rem)],
                        acc.at[pl.ds(rbase + (RPT // CH) * CH, rem)])

    def _issue_idx(bufs, semi, b):
        si, di = bufs[0], bufs[1]
        pltpu.async_copy(src_hbm.at[pl.ds(b, CH)], si, semi)
        pltpu.async_copy(dst_hbm.at[pl.ds(b, CH)], di, semi)

    def _wait_idx(bufs, semi):
        si, di = bufs[0], bufs[1]
        pltpu.make_async_copy(src_hbm.at[pl.ds(0, CH)], si, semi).wait()
        pltpu.make_async_copy(dst_hbm.at[pl.ds(0, CH)], di, semi).wait()

    def _issue(bufs, sem):
        si, di, pb, qb, hb, _ = bufs
        pltpu.async_copy(p_hbm.at[si], pb, sem)
        pltpu.async_copy(q_hbm.at[di], qb, sem)
        pltpu.async_copy(h_hbm.at[si], hb, sem)

    def _wait(bufs, sem):
        si, di, pb, qb, hb, _ = bufs
        pltpu.make_async_copy(p_hbm.at[si], pb, sem).wait()
        pltpu.make_async_copy(q_hbm.at[di], qb, sem).wait()
        pltpu.make_async_copy(h_hbm.at[si], hb, sem).wait()

    def _compute(bufs):
        _, _, pb, qb, hb, sb = bufs

        @plsc.parallel_loop(0, CH, unroll=2)
        def _edge(e):
            z = pb[e] + qb[e]
            w = jnp.exp(jnp.where(z >= 0, z, 0.2 * z))
            sb[e, pl.ds(F1, 16)] = w
            for hh in range(HEADS):
                wb = w.at[jnp.full((16,), hh, jnp.int32)].get(
                    mode="promise_in_bounds")
                sb[e, pl.ds(hh * HID, HID)] = (
                    wb * hb[e, pl.ds(hh * HID, HID)])

    def _scatter(bufs):
        _, di, _, _, _, sb = bufs
        pltpu.sync_copy(sb, acc.at[di], add=True)

    ebase = wid * EPW
    pltpu.sync_copy(src_hbm.at[pl.ds(ebase, CH)], sidx0)
    pltpu.sync_copy(dst_hbm.at[pl.ds(ebase, CH)], didx0)
    _issue(B0, sem0)
    _issue_idx(B1, semi1, ebase + CH)
    plsc.subcore_barrier()

    NPAIR = (NCHUNK - 1) // 2

    def _pair(k, _):
        b = ebase + (2 * k) * CH
        _wait(B0, sem0)
        _wait_idx(B1, semi1)
        _issue(B1, sem1)
        _compute(B0)
        _scatter(B0)
        _issue_idx(B0, semi0, b + 2 * CH)
        _wait(B1, sem1)
        _wait_idx(B0, semi0)
        _issue(B0, sem0)
        _compute(B1)
        _scatter(B1)

        @pl.when(k < NPAIR - 1)
        def _():
            _issue_idx(B1, semi1, b + 3 * CH)
        return 0
    lax.fori_loop(0, NPAIR, _pair, 0)
    _wait(B0, sem0)
    _compute(B0)
    _scatter(B0)

    plsc.subcore_barrier()
    for k in range(RPT // 125):
        r0 = rbase + k * 125
        pltpu.sync_copy(acc.at[pl.ds(r0, 125)], out_hbm.at[c, pl.ds(r0, 125)])


def _sc1(src, dst, p, q, h):
    mesh = plsc.VectorSubcoreMesh(core_axis_name="c", subcore_axis_name="s")
    f = functools.partial(
        pl.kernel,
        out_type=jax.ShapeDtypeStruct((NC, N, ROW1), jnp.float32),
        mesh=mesh,
        scratch_types=(
            2 * [
                pltpu.VMEM((CH,), jnp.int32),
                pltpu.VMEM((CH,), jnp.int32),
                pltpu.VMEM((CH, 16), jnp.float32),
                pltpu.VMEM((CH, 16), jnp.float32),
                pltpu.VMEM((CH, F1), jnp.float32),
            ] + [
                pltpu.VMEM((CH, ROW1), jnp.float32),
                pltpu.VMEM_SHARED((N, ROW1), jnp.float32),
                pltpu.SemaphoreType.DMA,
                pltpu.SemaphoreType.DMA,
                pltpu.SemaphoreType.DMA,
                pltpu.SemaphoreType.DMA,
            ]
        ),
        compiler_params=pltpu.CompilerParams(use_tc_tiling_on_sc=False, needs_layout_passes=False),
    )(_sc1_body)
    return f(src, dst, p, q, h)


def _tcb_body(acc_ref, self_ref, b1_ref, w2_ref, as2_ref, ad2_ref, r_ref,
              h2_ref, asv_ref, adv_ref, self2_ref):
    tot = acc_ref[0] + acc_ref[1] + self_ref[...]
    num = tot[:, :F1]
    den = tot[:, F1:F1 + HEADS]
    den128 = jnp.dot(den, r_ref[...], preferred_element_type=jnp.float32)
    h1 = num / (den128 + 1e-16) + b1_ref[...]
    h1 = jnp.where(h1 > 0, h1, jnp.exp(h1) - 1.0)
    h2 = jnp.dot(h1, w2_ref[...], preferred_element_type=jnp.float32)
    a_s = jnp.dot(h2, as2_ref[...], preferred_element_type=jnp.float32)
    a_d = jnp.dot(h2, ad2_ref[...], preferred_element_type=jnp.float32)
    z = a_s + a_d
    ws = jnp.exp(jnp.where(z >= 0, z, 0.2 * z))
    opad = jnp.ones((_TCB, F2PAD - OUT_DIM), jnp.float32)
    hp = jnp.concatenate([h2, opad], axis=1)
    h2_ref[...] = hp
    asv_ref[...] = a_s
    adv_ref[...] = a_d
    self2_ref[...] = hp * ws


def _tcb(acc, selfacc, b1, w2, as2, ad2, rep):
    return pl.pallas_call(
        _tcb_body,
        grid=(_TCG,),
        in_specs=[
            pl.BlockSpec((NC, _TCB, ROW1), lambda i: (0, i, 0)),
            pl.BlockSpec((_TCB, ROW1), lambda i: (i, 0)),
            pl.BlockSpec((1, F1), lambda i: (0, 0)),
            pl.BlockSpec((F1, OUT_DIM), lambda i: (0, 0)),
            pl.BlockSpec((OUT_DIM, 1), lambda i: (0, 0)),
            pl.BlockSpec((OUT_DIM, 1), lambda i: (0, 0)),
            pl.BlockSpec((HEADS, F1), lambda i: (0, 0)),
        ],
        out_specs=[
            pl.BlockSpec((_TCB, F2PAD), lambda i: (i, 0)),
            pl.BlockSpec((_TCB, 1), lambda i: (i, 0)),
            pl.BlockSpec((_TCB, 1), lambda i: (i, 0)),
            pl.BlockSpec((_TCB, ROW2), lambda i: (i, 0)),
        ],
        out_shape=[
            jax.ShapeDtypeStruct((N, F2PAD), jnp.float32),
            jax.ShapeDtypeStruct((N, 1), jnp.float32),
            jax.ShapeDtypeStruct((N, 1), jnp.float32),
            jax.ShapeDtypeStruct((N, ROW2), jnp.float32),
        ],
    )(acc, selfacc, b1, w2, as2, ad2, rep)


def _sc2_body(src_hbm, dst_hbm, h2_hbm, as_hbm, ad_hbm, out_hbm,
              sidx0, didx0, hbuf0, sbuf0,
              sidx1, didx1, hbuf1, sbuf1,
              wbuf, asb, adb, acc, sem0, sem1, semi0, semi1):
    c = lax.axis_index("c")
    s = lax.axis_index("s")
    wid = c * NS + s
    B0 = (sidx0, didx0, hbuf0, sbuf0)
    B1 = (sidx1, didx1, hbuf1, sbuf1)

    def _zrow(r, _):
        for j in range(ROW2 // 16):
            sbuf0[r, pl.ds(j * 16, 16)] = jnp.zeros((16,), jnp.float32)
        return 0
    lax.fori_loop(0, CH, _zrow, 0)
    rbase = s * RPT
    for k in range(RPT // CH):
        pltpu.sync_copy(sbuf0, acc.at[pl.ds(rbase + k * CH, CH)])
    rem = RPT - (RPT // CH) * CH
    if rem:
        pltpu.sync_copy(sbuf0.at[pl.ds(0, rem)],
                        acc.at[pl.ds(rbase + (RPT // CH) * CH, rem)])

    pltpu.sync_copy(as_hbm, asb)
    pltpu.sync_copy(ad_hbm, adb)

    def _issue_idx(bufs, semi, b):
        si, di = bufs[0], bufs[1]
        pltpu.async_copy(src_hbm.at[pl.ds(b, CH)], si, semi)
        pltpu.async_copy(dst_hbm.at[pl.ds(b, CH)], di, semi)

    def _wait_idx(bufs, semi):
        si, di = bufs[0], bufs[1]
        pltpu.make_async_copy(src_hbm.at[pl.ds(0, CH)], si, semi).wait()
        pltpu.make_async_copy(dst_hbm.at[pl.ds(0, CH)], di, semi).wait()

    def _issue(bufs, sem):
        si, di, hb, _ = bufs
        pltpu.async_copy(h2_hbm.at[si], hb, sem)

    def _wait(bufs, sem):
        si, _, hb, _ = bufs
        pltpu.make_async_copy(h2_hbm.at[si], hb, sem).wait()

    def _compute(bufs):
        si, di, hb, sb = bufs

        @plsc.parallel_loop(0, CH // 16, unroll=2)
        def _att(k):
            sv = si[pl.ds(k * 16, 16)]
            dv = di[pl.ds(k * 16, 16)]
            z = plsc.load_gather(asb, [sv]) + plsc.load_gather(adb, [dv])
            wbuf[pl.ds(k * 16, 16)] = jnp.exp(jnp.where(z >= 0, z, 0.2 * z))

        @plsc.parallel_loop(0, CH // 16, unroll=2)
        def _grp(k):
            w16 = wbuf[pl.ds(k * 16, 16)]
            for j in range(16):
                e = k * 16 + j
                wb = w16.at[jnp.full((16,), j, jnp.int32)].get(
                    mode="promise_in_bounds")
                for t in range(F2PAD // 16):
                    sb[e, pl.ds(t * 16, 16)] = (
                        wb * hb[e, pl.ds(t * 16, 16)])

    def _scatter(bufs):
        _, di, _, sb = bufs
        pltpu.sync_copy(sb, acc.at[di], add=True)

    ebase = wid * EPW
    pltpu.sync_copy(src_hbm.at[pl.ds(ebase, CH)], sidx0)
    pltpu.sync_copy(dst_hbm.at[pl.ds(ebase, CH)], didx0)
    _issue(B0, sem0)
    _issue_idx(B1, semi1, ebase + CH)
    plsc.subcore_barrier()

    NPAIR = (NCHUNK - 1) // 2

    def _pair(k, _):
        b = ebase + (2 * k) * CH
        _wait(B0, sem0)
        _wait_idx(B1, semi1)
        _issue(B1, sem1)
        _compute(B0)
        _scatter(B0)
        _issue_idx(B0, semi0, b + 2 * CH)
        _wait(B1, sem1)
        _wait_idx(B0, semi0)
        _issue(B0, sem0)
        _compute(B1)
        _scatter(B1)

        @pl.when(k < NPAIR - 1)
        def _():
            _issue_idx(B1, semi1, b + 3 * CH)
        return 0
    lax.fori_loop(0, NPAIR, _pair, 0)
    _wait(B0, sem0)
    _compute(B0)
    _scatter(B0)

    plsc.subcore_barrier()
    for k in range(RPT // 125):
        r0 = rbase + k * 125
        pltpu.sync_copy(acc.at[pl.ds(r0, 125)], out_hbm.at[c, pl.ds(r0, 125)])


def _sc2(src, dst, h2, as2, ad2):
    mesh = plsc.VectorSubcoreMesh(core_axis_name="c", subcore_axis_name="s")
    f = functools.partial(
        pl.kernel,
        out_type=jax.ShapeDtypeStruct((NC, N, ROW2), jnp.float32),
        mesh=mesh,
        scratch_types=(
            2 * [
                pltpu.VMEM((CH,), jnp.int32),
                pltpu.VMEM((CH,), jnp.int32),
                pltpu.VMEM((CH, F2PAD), jnp.float32),
                pltpu.VMEM((CH, ROW2), jnp.float32),
            ] + [
                pltpu.VMEM((CH,), jnp.float32),
                pltpu.VMEM((N,), jnp.float32),
                pltpu.VMEM((N,), jnp.float32),
                pltpu.VMEM_SHARED((N, ROW2), jnp.float32),
                pltpu.SemaphoreType.DMA,
                pltpu.SemaphoreType.DMA,
                pltpu.SemaphoreType.DMA,
                pltpu.SemaphoreType.DMA,
            ]
        ),
        compiler_params=pltpu.CompilerParams(use_tc_tiling_on_sc=False, needs_layout_passes=False),
    )(_sc2_body)
    return f(src, dst, h2, as2, ad2)


def _tcc_body(acc_ref, self_ref, b2_ref, out_ref):
    tot = acc_ref[0] + acc_ref[1] + self_ref[...]
    num = tot[:, :OUT_DIM]
    den = tot[:, OUT_DIM:OUT_DIM + 1]
    out_ref[...] = num / (den + 1e-16) + b2_ref[...]


def _tcc(acc, selfacc, b2):
    return pl.pallas_call(
        _tcc_body,
        grid=(_TCG,),
        in_specs=[
            pl.BlockSpec((NC, _TCB, ROW2), lambda i: (0, i, 0)),
            pl.BlockSpec((_TCB, ROW2), lambda i: (i, 0)),
            pl.BlockSpec((1, OUT_DIM), lambda i: (0, 0)),
        ],
        out_specs=pl.BlockSpec((_TCB, OUT_DIM), lambda i: (i, 0)),
        out_shape=jax.ShapeDtypeStruct((N, OUT_DIM), jnp.float32),
    )(acc, selfacc, b2)


@jax.jit
def kernel(x, edge_index, W1, a_src1, a_dst1, b1, W2, a_src2, a_dst2, b2):
    src = edge_index[0]
    dst = edge_index[1]

    eye = jnp.eye(HEADS, dtype=jnp.float32)
    as_mat = (eye[:, None, :] * a_src1[:, :, None]).reshape(F1, HEADS)
    ad_mat = (eye[:, None, :] * a_dst1[:, :, None]).reshape(F1, HEADS)
    rep = jnp.repeat(eye, HID, axis=1)

    h, p, q, selfacc = _tca(x, W1, as_mat, ad_mat, rep)
    acc1 = _sc1(src, dst, p, q, h)
    h2, asv, adv, selfacc2 = _tcb(acc1, selfacc, b1.reshape(1, F1), W2,
                                  a_src2.reshape(OUT_DIM, 1),
                                  a_dst2.reshape(OUT_DIM, 1), rep)
    acc2 = _sc2(src, dst, h2, asv.reshape(N), adv.reshape(N))
    return _tcc(acc2, selfacc2, b2.reshape(1, OUT_DIM))


# speedup vs baseline: 2.4951x; 1.1007x over previous
import functools
import jax
import jax.numpy as jnp
from jax import lax
from jax.experimental import pallas as pl
from jax.experimental.pallas import tpu as pltpu
from jax.experimental.pallas import tpu_sc as plsc

N = 10000
E = 320000
IN_DIM = 128
HID = 16
HEADS = 8
F1 = HEADS * HID
OUT_DIM = 40
F2PAD = 48
ROW1 = F1 + 16
ROW2 = F2PAD

NC = 2
NS = 16
NW = NC * NS
EPW = E // NW
CH = 80
NCHUNK = EPW // CH
RPT = N // NS

_TCB = 2000
_TCG = N // _TCB


def _tca_body(x_ref, w1_ref, as_ref, ad_ref, r_ref, h_ref, p_ref, q_ref,
              self_ref):
    h = jnp.dot(x_ref[...], w1_ref[...], preferred_element_type=jnp.float32)
    a_s = jnp.dot(h, as_ref[...], preferred_element_type=jnp.float32)
    a_d = jnp.dot(h, ad_ref[...], preferred_element_type=jnp.float32)
    z = a_s + a_d
    ws = jnp.exp(jnp.where(z >= 0, z, 0.2 * z))
    numself = h * jnp.dot(ws, r_ref[...], preferred_element_type=jnp.float32)
    h_ref[...] = h
    p_ref[...] = jnp.concatenate([a_s, a_s], axis=1)
    q_ref[...] = jnp.concatenate([a_d, a_d], axis=1)
    self_ref[...] = jnp.concatenate([numself, ws, ws], axis=1)


def _tca(x, w1, as_mat, ad_mat, rep):
    return pl.pallas_call(
        _tca_body,
        grid=(_TCG,),
        in_specs=[
            pl.BlockSpec((_TCB, IN_DIM), lambda i: (i, 0)),
            pl.BlockSpec((IN_DIM, F1), lambda i: (0, 0)),
            pl.BlockSpec((F1, HEADS), lambda i: (0, 0)),
            pl.BlockSpec((F1, HEADS), lambda i: (0, 0)),
            pl.BlockSpec((HEADS, F1), lambda i: (0, 0)),
        ],
        out_specs=[
            pl.BlockSpec((_TCB, F1), lambda i: (i, 0)),
            pl.BlockSpec((_TCB, 16), lambda i: (i, 0)),
            pl.BlockSpec((_TCB, 16), lambda i: (i, 0)),
            pl.BlockSpec((_TCB, ROW1), lambda i: (i, 0)),
        ],
        out_shape=[
            jax.ShapeDtypeStruct((N, F1), jnp.float32),
            jax.ShapeDtypeStruct((N, 16), jnp.float32),
            jax.ShapeDtypeStruct((N, 16), jnp.float32),
            jax.ShapeDtypeStruct((N, ROW1), jnp.float32),
        ],
    )(x, w1, as_mat, ad_mat, rep)


def _sc1_body(src_hbm, dst_hbm, p_hbm, q_hbm, h_hbm, out_hbm,
              sidx0, didx0, pbuf0, qbuf0, hbuf0,
              sidx1, didx1, pbuf1, qbuf1, hbuf1,
              sbuf0, acc, sem0, sem1, semi0, semi1, sems0, sems1):
    c = lax.axis_index("c")
    s = lax.axis_index("s")
    wid = c * NS + s
    B0 = (sidx0, didx0, pbuf0, qbuf0, hbuf0, sbuf0)
    B1 = (sidx1, didx1, pbuf1, qbuf1, hbuf1, sbuf0)

    def _zrow(r, _):
        for j in range(ROW1 // 16):
            sbuf0[r, pl.ds(j * 16, 16)] = jnp.zeros((16,), jnp.float32)
        return 0
    lax.fori_loop(0, CH, _zrow, 0)
    rbase = s * RPT
    for k in range(RPT // CH):
        pltpu.sync_copy(sbuf0, acc.at[pl.ds(rbase + k * CH, CH)])
    rem = RPT - (RPT // CH) * CH
    if rem:
        pltpu.sync_copy(sbuf0.at[pl.ds(0, rem)],
                        acc.at[pl.ds(rbase + (RPT // CH) * CH, rem)])

    def _issue_idx(bufs, semi, b):
        si, di = bufs[0], bufs[1]
        pltpu.async_copy(src_hbm.at[pl.ds(b, CH)], si, semi)
        pltpu.async_copy(dst_hbm.at[pl.ds(b, CH)], di, semi)

    def _wait_idx(bufs, semi):
        si, di = bufs[0], bufs[1]
        pltpu.make_async_copy(src_hbm.at[pl.ds(0, CH)], si, semi).wait()
        pltpu.make_async_copy(dst_hbm.at[pl.ds(0, CH)], di, semi).wait()

    def _issue(bufs, sem):
        si, di, pb, qb, hb, _ = bufs
        pltpu.async_copy(p_hbm.at[si], pb, sem)
        pltpu.async_copy(q_hbm.at[di], qb, sem)
        pltpu.async_copy(h_hbm.at[si], hb, sem)

    def _wait(bufs, sem):
        si, di, pb, qb, hb, _ = bufs
        pltpu.make_async_copy(p_hbm.at[si], pb, sem).wait()
        pltpu.make_async_copy(q_hbm.at[di], qb, sem).wait()
        pltpu.make_async_copy(h_hbm.at[si], hb, sem).wait()

    def _compute(bufs):
        _, _, pb, qb, hb, sb = bufs

        @plsc.parallel_loop(0, CH, unroll=2)
        def _edge(e):
            z = pb[e] + qb[e]
            w = jnp.exp(jnp.where(z >= 0, z, 0.2 * z))
            sb[e, pl.ds(F1, 16)] = w
            for hh in range(HEADS):
                wb = w.at[jnp.full((16,), hh, jnp.int32)].get(
                    mode="promise_in_bounds")
                sb[e, pl.ds(hh * HID, HID)] = (
                    wb * hb[e, pl.ds(hh * HID, HID)])

    def _scatter(bufs, sems):
        _, di, _, _, _, sb = bufs
        pltpu.async_copy(sb, acc.at[di], sems, add=True)

    def _wait_scatter(bufs, sems):
        _, di, _, _, _, sb = bufs
        pltpu.make_async_copy(sb, acc.at[di], sems).wait()

    ebase = wid * EPW
    pltpu.sync_copy(src_hbm.at[pl.ds(ebase, CH)], sidx0)
    pltpu.sync_copy(dst_hbm.at[pl.ds(ebase, CH)], didx0)
    _issue(B0, sem0)
    _issue_idx(B1, semi1, ebase + CH)
    plsc.subcore_barrier()

    NPAIR = (NCHUNK - 1) // 2

    def _pair(k, _):
        b = ebase + (2 * k) * CH
        _wait(B0, sem0)
        _wait_idx(B1, semi1)
        _issue(B1, sem1)

        @pl.when(k > 0)
        def _():
            _wait_scatter(B1, sems1)
        _compute(B0)
        _scatter(B0, sems0)
        _issue_idx(B0, semi0, b + 2 * CH)
        _wait(B1, sem1)
        _wait_idx(B0, semi0)
        _issue(B0, sem0)
        _wait_scatter(B0, sems0)
        _compute(B1)
        _scatter(B1, sems1)

        @pl.when(k < NPAIR - 1)
        def _():
            _issue_idx(B1, semi1, b + 3 * CH)
        return 0
    lax.fori_loop(0, NPAIR, _pair, 0)
    _wait(B0, sem0)
    _wait_scatter(B1, sems1)
    _compute(B0)
    _scatter(B0, sems0)
    _wait_scatter(B0, sems0)

    plsc.subcore_barrier()
    for k in range(RPT // 125):
        r0 = rbase + k * 125
        pltpu.sync_copy(acc.at[pl.ds(r0, 125)], out_hbm.at[c, pl.ds(r0, 125)])


def _sc1(src, dst, p, q, h):
    mesh = plsc.VectorSubcoreMesh(core_axis_name="c", subcore_axis_name="s")
    f = functools.partial(
        pl.kernel,
        out_type=jax.ShapeDtypeStruct((NC, N, ROW1), jnp.float32),
        mesh=mesh,
        scratch_types=(
            2 * [
                pltpu.VMEM((CH,), jnp.int32),
                pltpu.VMEM((CH,), jnp.int32),
                pltpu.VMEM((CH, 16), jnp.float32),
                pltpu.VMEM((CH, 16), jnp.float32),
                pltpu.VMEM((CH, F1), jnp.float32),
            ] + [
                pltpu.VMEM((CH, ROW1), jnp.float32),
                pltpu.VMEM_SHARED((N, ROW1), jnp.float32),
                pltpu.SemaphoreType.DMA,
                pltpu.SemaphoreType.DMA,
                pltpu.SemaphoreType.DMA,
                pltpu.SemaphoreType.DMA,
                pltpu.SemaphoreType.DMA,
                pltpu.SemaphoreType.DMA,
            ]
        ),
        compiler_params=pltpu.CompilerParams(use_tc_tiling_on_sc=False, needs_layout_passes=False),
    )(_sc1_body)
    return f(src, dst, p, q, h)


def _tcb_body(acc_ref, self_ref, b1_ref, w2_ref, as2_ref, ad2_ref, r_ref,
              h2_ref, asv_ref, adv_ref, self2_ref):
    tot = acc_ref[0] + acc_ref[1] + self_ref[...]
    num = tot[:, :F1]
    den = tot[:, F1:F1 + HEADS]
    den128 = jnp.dot(den, r_ref[...], preferred_element_type=jnp.float32)
    h1 = num / (den128 + 1e-16) + b1_ref[...]
    h1 = jnp.where(h1 > 0, h1, jnp.exp(h1) - 1.0)
    h2 = jnp.dot(h1, w2_ref[...], preferred_element_type=jnp.float32)
    a_s = jnp.dot(h2, as2_ref[...], preferred_element_type=jnp.float32)
    a_d = jnp.dot(h2, ad2_ref[...], preferred_element_type=jnp.float32)
    z = a_s + a_d
    ws = jnp.exp(jnp.where(z >= 0, z, 0.2 * z))
    opad = jnp.ones((_TCB, F2PAD - OUT_DIM), jnp.float32)
    hp = jnp.concatenate([h2, opad], axis=1)
    h2_ref[...] = hp
    asv_ref[...] = a_s
    adv_ref[...] = a_d
    self2_ref[...] = hp * ws


def _tcb(acc, selfacc, b1, w2, as2, ad2, rep):
    return pl.pallas_call(
        _tcb_body,
        grid=(_TCG,),
        in_specs=[
            pl.BlockSpec((NC, _TCB, ROW1), lambda i: (0, i, 0)),
            pl.BlockSpec((_TCB, ROW1), lambda i: (i, 0)),
            pl.BlockSpec((1, F1), lambda i: (0, 0)),
            pl.BlockSpec((F1, OUT_DIM), lambda i: (0, 0)),
            pl.BlockSpec((OUT_DIM, 1), lambda i: (0, 0)),
            pl.BlockSpec((OUT_DIM, 1), lambda i: (0, 0)),
            pl.BlockSpec((HEADS, F1), lambda i: (0, 0)),
        ],
        out_specs=[
            pl.BlockSpec((_TCB, F2PAD), lambda i: (i, 0)),
            pl.BlockSpec((_TCB, 1), lambda i: (i, 0)),
            pl.BlockSpec((_TCB, 1), lambda i: (i, 0)),
            pl.BlockSpec((_TCB, ROW2), lambda i: (i, 0)),
        ],
        out_shape=[
            jax.ShapeDtypeStruct((N, F2PAD), jnp.float32),
            jax.ShapeDtypeStruct((N, 1), jnp.float32),
            jax.ShapeDtypeStruct((N, 1), jnp.float32),
            jax.ShapeDtypeStruct((N, ROW2), jnp.float32),
        ],
    )(acc, selfacc, b1, w2, as2, ad2, rep)


def _sc2_body(src_hbm, dst_hbm, h2_hbm, as_hbm, ad_hbm, out_hbm,
              sidx0, didx0, hbuf0, sbuf0,
              sidx1, didx1, hbuf1, sbuf1,
              wbuf, asb, adb, acc, sem0, sem1, semi0, semi1, sems0, sems1):
    c = lax.axis_index("c")
    s = lax.axis_index("s")
    wid = c * NS + s
    B0 = (sidx0, didx0, hbuf0, sbuf0)
    B1 = (sidx1, didx1, hbuf1, sbuf1)

    def _zrow(r, _):
        for j in range(ROW2 // 16):
            sbuf0[r, pl.ds(j * 16, 16)] = jnp.zeros((16,), jnp.float32)
        return 0
    lax.fori_loop(0, CH, _zrow, 0)
    rbase = s * RPT
    for k in range(RPT // CH):
        pltpu.sync_copy(sbuf0, acc.at[pl.ds(rbase + k * CH, CH)])
    rem = RPT - (RPT // CH) * CH
    if rem:
        pltpu.sync_copy(sbuf0.at[pl.ds(0, rem)],
                        acc.at[pl.ds(rbase + (RPT // CH) * CH, rem)])

    pltpu.sync_copy(as_hbm, asb)
    pltpu.sync_copy(ad_hbm, adb)

    def _issue_idx(bufs, semi, b):
        si, di = bufs[0], bufs[1]
        pltpu.async_copy(src_hbm.at[pl.ds(b, CH)], si, semi)
        pltpu.async_copy(dst_hbm.at[pl.ds(b, CH)], di, semi)

    def _wait_idx(bufs, semi):
        si, di = bufs[0], bufs[1]
        pltpu.make_async_copy(src_hbm.at[pl.ds(0, CH)], si, semi).wait()
        pltpu.make_async_copy(dst_hbm.at[pl.ds(0, CH)], di, semi).wait()

    def _issue(bufs, sem):
        si, di, hb, _ = bufs
        pltpu.async_copy(h2_hbm.at[si], hb, sem)

    def _wait(bufs, sem):
        si, _, hb, _ = bufs
        pltpu.make_async_copy(h2_hbm.at[si], hb, sem).wait()

    def _compute(bufs):
        si, di, hb, sb = bufs

        @plsc.parallel_loop(0, CH // 16, unroll=2)
        def _att(k):
            sv = si[pl.ds(k * 16, 16)]
            dv = di[pl.ds(k * 16, 16)]
            z = plsc.load_gather(asb, [sv]) + plsc.load_gather(adb, [dv])
            wbuf[pl.ds(k * 16, 16)] = jnp.exp(jnp.where(z >= 0, z, 0.2 * z))

        @plsc.parallel_loop(0, CH // 16, unroll=2)
        def _grp(k):
            w16 = wbuf[pl.ds(k * 16, 16)]
            for j in range(16):
                e = k * 16 + j
                wb = w16.at[jnp.full((16,), j, jnp.int32)].get(
                    mode="promise_in_bounds")
                for t in range(F2PAD // 16):
                    sb[e, pl.ds(t * 16, 16)] = (
                        wb * hb[e, pl.ds(t * 16, 16)])

    def _scatter(bufs, sems):
        _, di, _, sb = bufs
        pltpu.async_copy(sb, acc.at[di], sems, add=True)

    def _wait_scatter(bufs, sems):
        _, di, _, sb = bufs
        pltpu.make_async_copy(sb, acc.at[di], sems).wait()

    ebase = wid * EPW
    pltpu.sync_copy(src_hbm.at[pl.ds(ebase, CH)], sidx0)
    pltpu.sync_copy(dst_hbm.at[pl.ds(ebase, CH)], didx0)
    _issue(B0, sem0)
    _issue_idx(B1, semi1, ebase + CH)
    plsc.subcore_barrier()

    NPAIR = (NCHUNK - 1) // 2

    def _pair(k, _):
        b = ebase + (2 * k) * CH
        _wait(B0, sem0)
        _wait_idx(B1, semi1)
        _issue(B1, sem1)

        @pl.when(k > 0)
        def _():
            _wait_scatter(B0, sems0)
        _compute(B0)
        _scatter(B0, sems0)
        _issue_idx(B0, semi0, b + 2 * CH)
        _wait(B1, sem1)
        _wait_idx(B0, semi0)
        _issue(B0, sem0)

        @pl.when(k > 0)
        def _():
            _wait_scatter(B1, sems1)
        _compute(B1)
        _scatter(B1, sems1)

        @pl.when(k < NPAIR - 1)
        def _():
            _issue_idx(B1, semi1, b + 3 * CH)
        return 0
    lax.fori_loop(0, NPAIR, _pair, 0)
    _wait(B0, sem0)
    _wait_scatter(B0, sems0)
    _wait_scatter(B1, sems1)
    _compute(B0)
    _scatter(B0, sems0)
    _wait_scatter(B0, sems0)

    plsc.subcore_barrier()
    for k in range(RPT // 125):
        r0 = rbase + k * 125
        pltpu.sync_copy(acc.at[pl.ds(r0, 125)], out_hbm.at[c, pl.ds(r0, 125)])


def _sc2(src, dst, h2, as2, ad2):
    mesh = plsc.VectorSubcoreMesh(core_axis_name="c", subcore_axis_name="s")
    f = functools.partial(
        pl.kernel,
        out_type=jax.ShapeDtypeStruct((NC, N, ROW2), jnp.float32),
        mesh=mesh,
        scratch_types=(
            2 * [
                pltpu.VMEM((CH,), jnp.int32),
                pltpu.VMEM((CH,), jnp.int32),
                pltpu.VMEM((CH, F2PAD), jnp.float32),
                pltpu.VMEM((CH, ROW2), jnp.float32),
            ] + [
                pltpu.VMEM((CH,), jnp.float32),
                pltpu.VMEM((N,), jnp.float32),
                pltpu.VMEM((N,), jnp.float32),
                pltpu.VMEM_SHARED((N, ROW2), jnp.float32),
                pltpu.SemaphoreType.DMA,
                pltpu.SemaphoreType.DMA,
                pltpu.SemaphoreType.DMA,
                pltpu.SemaphoreType.DMA,
                pltpu.SemaphoreType.DMA,
                pltpu.SemaphoreType.DMA,
            ]
        ),
        compiler_params=pltpu.CompilerParams(use_tc_tiling_on_sc=False, needs_layout_passes=False),
    )(_sc2_body)
    return f(src, dst, h2, as2, ad2)


def _tcc_body(acc_ref, self_ref, b2_ref, out_ref):
    tot = acc_ref[0] + acc_ref[1] + self_ref[...]
    num = tot[:, :OUT_DIM]
    den = tot[:, OUT_DIM:OUT_DIM + 1]
    out_ref[...] = num / (den + 1e-16) + b2_ref[...]


def _tcc(acc, selfacc, b2):
    return pl.pallas_call(
        _tcc_body,
        grid=(_TCG,),
        in_specs=[
            pl.BlockSpec((NC, _TCB, ROW2), lambda i: (0, i, 0)),
            pl.BlockSpec((_TCB, ROW2), lambda i: (i, 0)),
            pl.BlockSpec((1, OUT_DIM), lambda i: (0, 0)),
        ],
        out_specs=pl.BlockSpec((_TCB, OUT_DIM), lambda i: (i, 0)),
        out_shape=jax.ShapeDtypeStruct((N, OUT_DIM), jnp.float32),
    )(acc, selfacc, b2)


@jax.jit
def kernel(x, edge_index, W1, a_src1, a_dst1, b1, W2, a_src2, a_dst2, b2):
    src = edge_index[0]
    dst = edge_index[1]

    eye = jnp.eye(HEADS, dtype=jnp.float32)
    as_mat = (eye[:, None, :] * a_src1[:, :, None]).reshape(F1, HEADS)
    ad_mat = (eye[:, None, :] * a_dst1[:, :, None]).reshape(F1, HEADS)
    rep = jnp.repeat(eye, HID, axis=1)

    h, p, q, selfacc = _tca(x, W1, as_mat, ad_mat, rep)
    acc1 = _sc1(src, dst, p, q, h)
    h2, asv, adv, selfacc2 = _tcb(acc1, selfacc, b1.reshape(1, F1), W2,
                                  a_src2.reshape(OUT_DIM, 1),
                                  a_dst2.reshape(OUT_DIM, 1), rep)
    acc2 = _sc2(src, dst, h2, asv.reshape(N), adv.reshape(N))
    return _tcc(acc2, selfacc2, b2.reshape(1, OUT_DIM))
